# baseline (speedup 1.0000x reference)
import jax
import jax.numpy as jnp
from jax.experimental import pallas as pl
from jax.experimental.pallas import tpu as pltpu

_EPS = 1e-5
_N_CLASS = 40


def _fused_hgnn_kernel(x_hbm, dvh_hbm, inv_hbm, wc_ref, w1_ref, b1_ref,
                       w2_ref, b2_ref, g1_ref, be1_ref, g2_ref, be2_ref,
                       out_ref, x_ref, dvh_ref, inv_ref, t_ref, y_ref,
                       sx0, sx1, si0, si1, sd0, sd1):
    f32 = jnp.float32
    n = x_ref.shape[0]
    m = inv_ref.shape[0]
    hn = n // 2
    hm = m // 2

    cps = [
        pltpu.make_async_copy(x_hbm.at[pl.ds(0, hn)], x_ref.at[pl.ds(0, hn)], sx0),
        pltpu.make_async_copy(x_hbm.at[pl.ds(hn, hn)], x_ref.at[pl.ds(hn, hn)], sx1),
        pltpu.make_async_copy(inv_hbm.at[pl.ds(0, hm)], inv_ref.at[pl.ds(0, hm)], si0),
        pltpu.make_async_copy(inv_hbm.at[pl.ds(hm, hm)], inv_ref.at[pl.ds(hm, hm)], si1),
        pltpu.make_async_copy(dvh_hbm.at[pl.ds(0, hn)], dvh_ref.at[pl.ds(0, hn)], sd0),
        pltpu.make_async_copy(dvh_hbm.at[pl.ds(hn, hn)], dvh_ref.at[pl.ds(hn, hn)], sd1),
    ]
    for cp in cps:
        cp.start()

    cps[0].wait()
    x0 = x_ref[pl.ds(0, hn), :]
    s0 = jnp.sum(x0, axis=0, keepdims=True)
    q0 = jnp.sum(x0 * x0, axis=0, keepdims=True)
    cps[1].wait()
    x1 = x_ref[pl.ds(hn, hn), :]
    s1 = jnp.sum(x1, axis=0, keepdims=True)
    q1 = jnp.sum(x1 * x1, axis=0, keepdims=True)
    mu1 = (s0 + s1) * (1.0 / n)
    var1 = (q0 + q1) * (1.0 / n) - mu1 * mu1
    scale1 = g1_ref[...] * jax.lax.rsqrt(var1 + _EPS)
    shift1 = be1_ref[...] - scale1 * mu1

    xbn = x_ref[...] * scale1 + shift1
    h1 = jnp.dot(xbn, w1_ref[...], preferred_element_type=f32) + b1_ref[...]

    cps[2].wait()
    t_ref[pl.ds(0, hm), :] = jnp.dot(inv_ref[pl.ds(0, hm), :], h1,
                                     preferred_element_type=f32)
    cps[3].wait()
    t_ref[pl.ds(hm, hm), :] = jnp.dot(inv_ref[pl.ds(hm, hm), :], h1,
                                      preferred_element_type=f32)
    tw = wc_ref[...] * t_ref[...]

    cps[4].wait()
    y_ref[pl.ds(0, hn), :] = jnp.dot(dvh_ref[pl.ds(0, hn), :], tw,
                                     preferred_element_type=f32)
    cps[5].wait()
    y_ref[pl.ds(hn, hn), :] = jnp.dot(dvh_ref[pl.ds(hn, hn), :], tw,
                                      preferred_element_type=f32)
    h = y_ref[...]

    mu2 = jnp.mean(h, axis=0, keepdims=True)
    hc = h - mu2
    var2 = jnp.mean(hc * hc, axis=0, keepdims=True)
    scale2 = g2_ref[...] * jax.lax.rsqrt(var2 + _EPS)
    r = jnp.maximum(hc * scale2 + be2_ref[...], 0.0)

    s3 = jnp.sum(r, axis=0, keepdims=True)
    q3 = jnp.sum(r * r, axis=0, keepdims=True)
    mu3 = s3 * (1.0 / n)
    var3 = q3 * (1.0 / n) - mu3 * mu3
    scale3 = g2_ref[...] * jax.lax.rsqrt(var3 + _EPS)
    r2 = r * scale3 + (be2_ref[...] - scale3 * mu3)

    u = jnp.dot(r2, w2_ref[...], preferred_element_type=f32) + b2_ref[...]
    t2 = jnp.dot(inv_ref[...], u, preferred_element_type=f32)
    tw2 = wc_ref[...] * t2
    out_ref[...] = jnp.dot(dvh_ref[...], tw2, preferred_element_type=f32)


def kernel(x, DV2_H, invDE_HT_DV2, W, W1, b1, W2, b2,
           bn1_gamma, bn1_beta, bn2_gamma, bn2_beta):
    n, in_ch = x.shape
    m = DV2_H.shape[1]
    n_hid = W1.shape[1]
    c_pad = 128

    W2p = jnp.zeros((n_hid, c_pad), dtype=W2.dtype).at[:, :_N_CLASS].set(W2)
    b2p = jnp.zeros((1, c_pad), dtype=b2.dtype).at[0, :_N_CLASS].set(b2)

    vmem = pl.BlockSpec(memory_space=pltpu.MemorySpace.VMEM)
    hbm = pl.BlockSpec(memory_space=pl.ANY)
    out = pl.pallas_call(
        _fused_hgnn_kernel,
        out_shape=jax.ShapeDtypeStruct((n, c_pad), jnp.float32),
        in_specs=[hbm, hbm, hbm] + [vmem] * 9,
        out_specs=vmem,
        scratch_shapes=[
            pltpu.VMEM((n, in_ch), jnp.float32),
            pltpu.VMEM((n, m), jnp.float32),
            pltpu.VMEM((m, n), jnp.float32),
            pltpu.VMEM((m, n_hid), jnp.float32),
            pltpu.VMEM((n, n_hid), jnp.float32),
        ] + [pltpu.SemaphoreType.DMA] * 6,
    )(
        x, DV2_H, invDE_HT_DV2,
        W.reshape(m, 1), W1, b1.reshape(1, n_hid),
        W2p, b2p,
        bn1_gamma.reshape(1, in_ch), bn1_beta.reshape(1, in_ch),
        bn2_gamma.reshape(1, n_hid), bn2_beta.reshape(1, n_hid),
    )
    return out[:, :_N_CLASS]
